# baseline (speedup 1.0000x reference)
import jax
import jax.numpy as jnp
from jax.experimental import pallas as pl
from jax.experimental.pallas import tpu as pltpu

LORA_ALPHA = 128.0


def _body(x_ref, wd_ref, wu_ref, o_ref, *, scale, r):
    xb = x_ref[...].astype(jnp.bfloat16)
    hx = jnp.dot(xb, wd_ref[...], preferred_element_type=jnp.float32)
    h = hx[:, :r]
    gs = hx[:, r:r + 1]
    h = h * (jax.nn.sigmoid(gs) * scale)
    o_ref[...] = jnp.dot(h.astype(jnp.bfloat16), wu_ref[...],
                         preferred_element_type=jnp.float32)


def kernel(x, W_down, W_up, input_gate):
    B, S, D_IN = x.shape
    R = W_down.shape[0]
    D_OUT = W_up.shape[0]
    scale = LORA_ALPHA / R

    T = B * S
    TILE = 512
    xf = x.reshape(T, D_IN)
    wide = max(128, R + 1)
    wd = jnp.zeros((D_IN, wide), jnp.float32)
    wd = wd.at[:, :R].set(W_down.T)
    wd = wd.at[:, R].set(input_gate[:, 0])
    wd = wd.astype(jnp.bfloat16)
    wu = W_up.T.astype(jnp.bfloat16)

    out = pl.pallas_call(
        lambda *refs: _body(*refs, scale=scale, r=R),
        grid=(T // TILE,),
        in_specs=[
            pl.BlockSpec((TILE, D_IN), lambda i: (i, 0)),
            pl.BlockSpec((D_IN, wide), lambda i: (0, 0)),
            pl.BlockSpec((R, D_OUT), lambda i: (0, 0)),
        ],
        out_specs=pl.BlockSpec((TILE, D_OUT), lambda i: (i, 0)),
        out_shape=jax.ShapeDtypeStruct((T, D_OUT), jnp.float32),
        compiler_params=pltpu.CompilerParams(
            dimension_semantics=("parallel",),
        ),
    )(xf, wd, wu)

    return out.reshape(B, S, D_OUT)


# speedup vs baseline: 1.0785x; 1.0785x over previous
import jax
import jax.numpy as jnp
from jax.experimental import pallas as pl
from jax.experimental.pallas import tpu as pltpu

LORA_ALPHA = 128.0


def _body(x_ref, wd_ref, wu_ref, o_ref, *, scale, r):
    xb = x_ref[...]
    hx = jnp.dot(xb, wd_ref[...], preferred_element_type=jnp.float32)
    h = hx[:, :r]
    gs = hx[:, r:r + 1]
    h = h * (jax.nn.sigmoid(gs) * scale)
    o_ref[...] = jnp.dot(h, wu_ref[...], preferred_element_type=jnp.float32)


def kernel(x, W_down, W_up, input_gate):
    B, S, D_IN = x.shape
    R = W_down.shape[0]
    D_OUT = W_up.shape[0]
    scale = LORA_ALPHA / R

    T = B * S
    TILE = 1024
    xf = x.reshape(T, D_IN)
    wide = max(128, R + 1)
    wd = jnp.zeros((D_IN, wide), jnp.float32)
    wd = wd.at[:, :R].set(W_down.T)
    wd = wd.at[:, R].set(input_gate[:, 0])
    wu = W_up.T

    out = pl.pallas_call(
        lambda *refs: _body(*refs, scale=scale, r=R),
        grid=(T // TILE,),
        in_specs=[
            pl.BlockSpec((TILE, D_IN), lambda i: (i, 0)),
            pl.BlockSpec((D_IN, wide), lambda i: (0, 0)),
            pl.BlockSpec((R, D_OUT), lambda i: (0, 0)),
        ],
        out_specs=pl.BlockSpec((TILE, D_OUT), lambda i: (i, 0)),
        out_shape=jax.ShapeDtypeStruct((T, D_OUT), jnp.float32),
        compiler_params=pltpu.CompilerParams(
            dimension_semantics=("parallel",),
        ),
    )(xf, wd, wu)

    return out.reshape(B, S, D_OUT)


# speedup vs baseline: 1.1588x; 1.0744x over previous
import jax
import jax.numpy as jnp
from jax.experimental import pallas as pl
from jax.experimental.pallas import tpu as pltpu

LORA_ALPHA = 128.0


def _body(xl_ref, xr_ref, g_ref, wd_ref, wu_ref, o_ref, *, scale, half):
    xl = xl_ref[...]
    xr = xr_ref[...]
    g = g_ref[...]
    gs = (jnp.sum(xl * g[:, :half], axis=-1, keepdims=True)
          + jnp.sum(xr * g[:, half:], axis=-1, keepdims=True))
    h = (jnp.dot(xl, wd_ref[:half, :], preferred_element_type=jnp.float32)
         + jnp.dot(xr, wd_ref[half:, :], preferred_element_type=jnp.float32))
    h = h * (jax.nn.sigmoid(gs) * scale)
    o_ref[...] = jnp.dot(h, wu_ref[...], preferred_element_type=jnp.float32)


def kernel(x, W_down, W_up, input_gate):
    B, S, D_IN = x.shape
    R = W_down.shape[0]
    D_OUT = W_up.shape[0]
    scale = LORA_ALPHA / R
    half = D_IN // 2

    T = B * S
    TILE = 1024
    xf = x.reshape(T, D_IN)
    wd = W_down.T
    wu = W_up.T
    g = input_gate.reshape(1, D_IN)

    out = pl.pallas_call(
        lambda *refs: _body(*refs, scale=scale, half=half),
        grid=(T // TILE,),
        in_specs=[
            pl.BlockSpec((TILE, half), lambda i: (i, 0)),
            pl.BlockSpec((TILE, half), lambda i: (i, 1)),
            pl.BlockSpec((1, D_IN), lambda i: (0, 0)),
            pl.BlockSpec((D_IN, R), lambda i: (0, 0)),
            pl.BlockSpec((R, D_OUT), lambda i: (0, 0)),
        ],
        out_specs=pl.BlockSpec((TILE, D_OUT), lambda i: (i, 0)),
        out_shape=jax.ShapeDtypeStruct((T, D_OUT), jnp.float32),
        compiler_params=pltpu.CompilerParams(
            dimension_semantics=("parallel",),
        ),
    )(xf, xf, g, wd, wu)

    return out.reshape(B, S, D_OUT)
